# Initial kernel scaffold; baseline (speedup 1.0000x reference)
import jax
import jax.numpy as jnp
from jax.experimental import pallas as pl


def kernel(features, edge_index, W1, b1, W2, b2, Wfc, bfc):
    raise NotImplementedError("write your pallas kernel here")


# speedup vs baseline: 5.4858x; 5.4858x over previous
import functools

import jax
import jax.numpy as jnp
from jax import lax
from jax.experimental import pallas as pl
from jax.experimental.pallas import tpu as pltpu
from jax.experimental.pallas import tpu_sc as plsc

N = 10000
NP = 10240
E = 160000
EP = 163840
D = 256
DH = 128
NCLS = 10
NC = 2
NS = 16
K = 128
NB = EP // (NS * K)
RPT = NP // NS
RB = 1024

_mesh = plsc.VectorSubcoreMesh(core_axis_name="c", subcore_axis_name="s")


@functools.partial(
    pl.kernel,
    out_type=jax.ShapeDtypeStruct((NC, NP), jnp.float32),
    mesh=_mesh,
    scratch_types=[
        pltpu.VMEM((NB, K), jnp.int32),
        pltpu.VMEM((K,), jnp.float32),
        pltpu.VMEM_SHARED((NP,), jnp.float32),
    ],
)
def _sc_degrees(edges_hbm, zero1_hbm, deg_out_hbm, idx_v, ones_v, deg_sh):
    c = lax.axis_index("c")
    s = lax.axis_index("s")
    row0 = s * RPT
    pltpu.sync_copy(zero1_hbm.at[pl.ds(row0, RPT)], deg_sh.at[pl.ds(row0, RPT)])
    pltpu.sync_copy(edges_hbm.at[c].at[s], idx_v)
    for i in range(K // 16):
        ones_v[pl.ds(i * 16, 16)] = jnp.ones((16,), jnp.float32)
    plsc.subcore_barrier()

    def body(j, carry):
        pltpu.sync_copy(ones_v, deg_sh.at[idx_v.at[j]], add=True)
        return carry

    lax.fori_loop(0, NB, body, 0)
    plsc.subcore_barrier()
    pltpu.sync_copy(deg_sh.at[pl.ds(row0, RPT)],
                    deg_out_hbm.at[c].at[pl.ds(row0, RPT)])


@functools.partial(
    pl.kernel,
    out_type=jax.ShapeDtypeStruct((NC, NP, DH), jnp.float32),
    mesh=_mesh,
    scratch_types=[
        pltpu.VMEM((NB, K), jnp.int32),
        pltpu.VMEM((NB, K), jnp.int32),
        pltpu.VMEM((K, DH), jnp.float32),
        pltpu.VMEM_SHARED((NP, DH), jnp.float32),
        pltpu.SemaphoreType.DMA,
    ],
)
def _sc_message(y_hbm, src_hbm, dst_hbm, zero2_hbm, agg_out_hbm,
                src_v, dst_v, rows_v, agg_sh, sem):
    c = lax.axis_index("c")
    s = lax.axis_index("s")
    row0 = s * RPT
    pltpu.sync_copy(zero2_hbm.at[pl.ds(row0, RPT)], agg_sh.at[pl.ds(row0, RPT)])
    pltpu.sync_copy(src_hbm.at[s], src_v)
    pltpu.sync_copy(dst_hbm.at[s], dst_v)
    plsc.subcore_barrier()

    def body(j, carry):
        pltpu.async_copy(y_hbm.at[c].at[src_v.at[j]], rows_v, sem).wait()
        pltpu.sync_copy(rows_v, agg_sh.at[dst_v.at[j]], add=True)
        return carry

    lax.fori_loop(0, NB, body, 0)
    plsc.subcore_barrier()
    pltpu.sync_copy(agg_sh.at[pl.ds(row0, RPT)],
                    agg_out_hbm.at[c].at[pl.ds(row0, RPT)])


_DOT = dict(preferred_element_type=jnp.float32,
            precision=jax.lax.Precision.HIGHEST)


def _tc1_body(deg_ref, x_ref, w_ref, y_ref, norm_ref):
    norm = lax.rsqrt(deg_ref[...] + 1.0)
    norm_ref[...] = norm
    xw = jnp.dot(x_ref[...], w_ref[...], **_DOT)
    y_ref[...] = (xw * norm[0][:, None])[None]


def _tc1(deg, x, w1):
    grid = (NP // RB, NC)
    return pl.pallas_call(
        _tc1_body,
        grid=grid,
        in_specs=[
            pl.BlockSpec((NC, RB), lambda i, j: (0, i)),
            pl.BlockSpec((RB, D), lambda i, j: (i, 0)),
            pl.BlockSpec((D, DH), lambda i, j: (0, j)),
        ],
        out_specs=[
            pl.BlockSpec((1, RB, DH), lambda i, j: (j, i, 0)),
            pl.BlockSpec((NC, RB), lambda i, j: (0, i)),
        ],
        out_shape=[
            jax.ShapeDtypeStruct((NC, NP, DH), jnp.float32),
            jax.ShapeDtypeStruct((NC, NP), jnp.float32),
        ],
    )(deg, x, w1)


def _tc2_body(agg_ref, y_ref, norm_ref, b_ref, w_ref, out_ref):
    agg = agg_ref[...]
    y = y_ref[...]
    norm = norm_ref[...]
    h = jnp.concatenate([agg[0] + y[0], agg[1] + y[1]], axis=1)
    h = jnp.maximum(h * norm[1][:, None] + b_ref[...], 0.0)
    hw = jnp.dot(h, w_ref[...], **_DOT)
    out_ref[...] = (hw * norm[0][:, None])[None]


def _tc2(agg, y, norm, b, w2):
    grid = (NP // RB, NC)
    return pl.pallas_call(
        _tc2_body,
        grid=grid,
        in_specs=[
            pl.BlockSpec((NC, RB, DH), lambda i, j: (0, i, 0)),
            pl.BlockSpec((NC, RB, DH), lambda i, j: (0, i, 0)),
            pl.BlockSpec((NC, RB), lambda i, j: (0, i)),
            pl.BlockSpec((1, D), lambda i, j: (0, 0)),
            pl.BlockSpec((D, DH), lambda i, j: (0, j)),
        ],
        out_specs=pl.BlockSpec((1, RB, DH), lambda i, j: (j, i, 0)),
        out_shape=jax.ShapeDtypeStruct((NC, NP, DH), jnp.float32),
    )(agg, y, norm, b, w2)


def _tc3_body(agg_ref, y_ref, norm_ref, b_ref, wfc_ref, bfc_ref, out_ref):
    agg = agg_ref[...]
    y = y_ref[...]
    norm = norm_ref[...]
    h = jnp.concatenate([agg[0] + y[0], agg[1] + y[1]], axis=1)
    h = jnp.maximum(h * norm[1][:, None] + b_ref[...], 0.0)
    out_ref[...] = jnp.dot(h, wfc_ref[...], **_DOT) + bfc_ref[...]


def _tc3(agg, y, norm, b, wfc, bfc):
    grid = (NP // RB,)
    return pl.pallas_call(
        _tc3_body,
        grid=grid,
        in_specs=[
            pl.BlockSpec((NC, RB, DH), lambda i: (0, i, 0)),
            pl.BlockSpec((NC, RB, DH), lambda i: (0, i, 0)),
            pl.BlockSpec((NC, RB), lambda i: (0, i)),
            pl.BlockSpec((1, D), lambda i: (0, 0)),
            pl.BlockSpec((D, NCLS), lambda i: (0, 0)),
            pl.BlockSpec((1, NCLS), lambda i: (0, 0)),
        ],
        out_specs=pl.BlockSpec((RB, NCLS), lambda i: (i, 0)),
        out_shape=jax.ShapeDtypeStruct((NP, NCLS), jnp.float32),
    )(agg, y, norm, b, wfc, bfc)


def kernel(features, edge_index, W1, b1, W2, b2, Wfc, bfc):
    ei = edge_index.astype(jnp.int32)
    pad = jnp.full((EP - E,), N, dtype=jnp.int32)
    src_r = jnp.concatenate([ei[0], pad]).reshape(NS, NB, K)
    dst_r = jnp.concatenate([ei[1], pad]).reshape(NS, NB, K)
    edges2 = jnp.stack([src_r, dst_r])

    xp = jnp.pad(features, ((0, NP - N), (0, 0)))
    z1 = jnp.zeros((NP,), jnp.float32)
    z2 = jnp.zeros((NP, DH), jnp.float32)

    deg = _sc_degrees(edges2, z1)
    y1, norm = _tc1(deg, xp, W1)
    agg1 = _sc_message(y1, src_r, dst_r, z2)
    y2 = _tc2(agg1, y1, norm, b1.reshape(1, D), W2)
    agg2 = _sc_message(y2, src_r, dst_r, z2)
    out = _tc3(agg2, y2, norm, b2.reshape(1, D), Wfc, bfc.reshape(1, NCLS))
    return out[:N]
